# baseline (speedup 1.0000x reference)
import functools

import jax
import jax.numpy as jnp
from jax import lax
from jax.experimental import pallas as pl
from jax.experimental.pallas import tpu as pltpu
from jax.experimental.pallas import tpu_sc as plsc

_VOCAB = 1000000
_EMBED = 64
_BATCH = 4096
_SEQ = 200
_N = _BATCH * _SEQ
_NC = 2
_NS = 16
_NW = _NC * _NS
_ROWS_W = _BATCH // _NW
_NB = 4
_L = 2
_GROUPS = _ROWS_W // _NB

_DB = 1000
_HV = _VOCAB // 2
_DG = _HV // _DB

_mesh = plsc.VectorSubcoreMesh(core_axis_name="c", subcore_axis_name="s")


def _depad_block(a_ref, b_ref, o_ref):
    o_ref[...] = jnp.concatenate([a_ref[...], b_ref[...]], axis=1)


_depad = pl.pallas_call(
    _depad_block,
    grid=(_DG,),
    in_specs=[
        pl.BlockSpec((_DB, _EMBED), lambda i: (i, 0)),
        pl.BlockSpec((_DB, _EMBED), lambda i: (i + _DG, 0)),
    ],
    out_specs=pl.BlockSpec((_DB, 2 * _EMBED), lambda i: (i, 0)),
    out_shape=jax.ShapeDtypeStruct((_HV, 2 * _EMBED), jnp.float32),
)


@functools.partial(
    pl.kernel,
    mesh=_mesh,
    compiler_params=pltpu.CompilerParams(use_tc_tiling_on_sc=False),
    out_type=jax.ShapeDtypeStruct((_N, 2 * _EMBED), jnp.float32),
    scratch_types=[
        pltpu.VMEM((_ROWS_W, _SEQ), jnp.int32),
        pltpu.VMEM((_NB, _SEQ, _EMBED), jnp.float32),
        pltpu.SemaphoreType.DMA((_NB,)),
        pltpu.SemaphoreType.DMA((_NB,)),
    ],
)
def _embed_lookup(x_hbm, table_hbm, out_hbm, idx_v, rows_v, gat_sem, out_sem):
    wid = lax.axis_index("s") * _NC + lax.axis_index("c")
    wrow = pl.multiple_of(wid * _ROWS_W, 8)
    pltpu.sync_copy(x_hbm.at[pl.ds(wrow, _ROWS_W)], idx_v)

    def start_gather(b, r):
        pltpu.make_async_copy(
            table_hbm.at[idx_v.at[r]], rows_v.at[b], gat_sem.at[b]
        ).start()

    def wait_gather(b):
        pltpu.make_async_copy(
            table_hbm.at[idx_v.at[0]], rows_v.at[b], gat_sem.at[b]
        ).wait()

    def start_out(b, r):
        base = pl.multiple_of((wrow + r) * _SEQ, 8)
        pltpu.make_async_copy(
            rows_v.at[b],
            out_hbm.at[pl.ds(base, _SEQ), pl.ds(0, _EMBED)],
            out_sem.at[b],
        ).start()

    def wait_out(b):
        pltpu.make_async_copy(
            rows_v.at[b],
            out_hbm.at[pl.ds(0, _SEQ), pl.ds(0, _EMBED)],
            out_sem.at[b],
        ).wait()

    def body(g, carry):
        for b in range(_NB):
            r = g * _NB + b
            @pl.when(g >= 1)
            def _():
                wait_out(b)

            start_gather(b, r)

            b2 = (b - _L) % _NB
            r2 = r - _L

            @pl.when(r2 >= 0)
            def _():
                wait_gather(b2)
                start_out(b2, r2)

        return carry

    lax.fori_loop(0, _GROUPS, body, 0)

    for k in range(_L):
        r2 = _ROWS_W - _L + k
        b2 = r2 % _NB
        wait_gather(b2)
        start_out(b2, r2)
    for b in range(_NB):
        wait_out(b)


def kernel(x, table):
    table_lin = _depad(table, table).reshape(_VOCAB, _EMBED)
    xj = 2 * (x % _HV) + x // _HV
    out = _embed_lookup(xj, table_lin)
    return out[:, :_EMBED].reshape(_BATCH, _SEQ, _EMBED)


# speedup vs baseline: 1.2061x; 1.2061x over previous
import functools

import jax
import jax.numpy as jnp
from jax import lax
from jax.experimental import pallas as pl
from jax.experimental.pallas import tpu as pltpu
from jax.experimental.pallas import tpu_sc as plsc

_VOCAB = 1000000
_EMBED = 64
_BATCH = 4096
_SEQ = 200
_N = _BATCH * _SEQ
_NC = 2
_NS = 16
_NW = _NC * _NS
_ROWS_W = _BATCH // _NW
_NB = 4
_L = 2
_GROUPS = _ROWS_W // _NB

_mesh = plsc.VectorSubcoreMesh(core_axis_name="c", subcore_axis_name="s")


@functools.partial(
    pl.kernel,
    mesh=_mesh,
    compiler_params=pltpu.CompilerParams(use_tc_tiling_on_sc=False),
    out_type=jax.ShapeDtypeStruct((_N, 2 * _EMBED), jnp.float32),
    scratch_types=[
        pltpu.VMEM((_ROWS_W, _SEQ), jnp.int32),
        pltpu.VMEM((_NB, _SEQ, _EMBED), jnp.float32),
        pltpu.SemaphoreType.DMA((_NB,)),
        pltpu.SemaphoreType.DMA((_NB,)),
    ],
)
def _embed_lookup(x_hbm, table_hbm, out_hbm, idx_v, rows_v, gat_sem, out_sem):
    wid = lax.axis_index("s") * _NC + lax.axis_index("c")
    wrow = pl.multiple_of(wid * _ROWS_W, 8)
    pltpu.sync_copy(x_hbm.at[pl.ds(wrow, _ROWS_W)], idx_v)

    def start_gather(b, r):
        pltpu.make_async_copy(
            table_hbm.at[idx_v.at[r]], rows_v.at[b], gat_sem.at[b]
        ).start()

    def wait_gather(b):
        pltpu.make_async_copy(
            table_hbm.at[idx_v.at[0]], rows_v.at[b], gat_sem.at[b]
        ).wait()

    def start_out(b, r):
        base = pl.multiple_of((wrow + r) * _SEQ, 8)
        pltpu.make_async_copy(
            rows_v.at[b],
            out_hbm.at[pl.ds(base, _SEQ), pl.ds(0, _EMBED)],
            out_sem.at[b],
        ).start()

    def wait_out(b):
        pltpu.make_async_copy(
            rows_v.at[b],
            out_hbm.at[pl.ds(0, _SEQ), pl.ds(0, _EMBED)],
            out_sem.at[b],
        ).wait()

    def body(g, carry):
        for b in range(_NB):
            r = g * _NB + b
            @pl.when(g >= 1)
            def _():
                wait_out(b)

            start_gather(b, r)

            b2 = (b - _L) % _NB
            r2 = r - _L

            @pl.when(r2 >= 0)
            def _():
                wait_gather(b2)
                start_out(b2, r2)

        return carry

    lax.fori_loop(0, _GROUPS, body, 0)

    for k in range(_L):
        r2 = _ROWS_W - _L + k
        b2 = r2 % _NB
        wait_gather(b2)
        start_out(b2, r2)
    for b in range(_NB):
        wait_out(b)


def kernel(x, table):
    out = _embed_lookup(x, table)
    return out[:, :_EMBED].reshape(_BATCH, _SEQ, _EMBED)
